# baseline (speedup 1.0000x reference)
import functools

import jax
import jax.numpy as jnp
from jax import lax
from jax.experimental import pallas as pl
from jax.experimental.pallas import tpu as pltpu
from jax.experimental.pallas import tpu_sc as plsc

NUM_USERS = 1000000
EMBED_DIM = 64
BATCH = 16384


@jax.jit
def _embed_lookup(user_inputs, table):
    info = plsc.get_sparse_core_info()
    nw = info.num_cores * info.num_subcores
    b_per_w = BATCH // nw
    mesh = plsc.VectorSubcoreMesh(core_axis_name="c", subcore_axis_name="s")

    table_pairs = table.reshape(NUM_USERS // 2, 2 * EMBED_DIM)
    idx = user_inputs.astype(jnp.int32)
    pair_idx = lax.shift_right_logical(idx, 1)

    @functools.partial(
        pl.kernel,
        mesh=mesh,
        out_type=jax.ShapeDtypeStruct((BATCH, 2 * EMBED_DIM), jnp.float32),
        scratch_types=[
            pltpu.VMEM((b_per_w,), jnp.int32),
            pltpu.VMEM((b_per_w, 2 * EMBED_DIM), jnp.float32),
            pltpu.SemaphoreType.DMA,
        ],
    )
    def gather_pairs(idx_hbm, tbl_hbm, out_hbm, idx_v, rows_v, sem):
        wid = lax.axis_index("s") * info.num_cores + lax.axis_index("c")
        base = wid * b_per_w
        pltpu.sync_copy(idx_hbm.at[pl.ds(base, b_per_w)], idx_v)
        pltpu.async_copy(tbl_hbm.at[idx_v], rows_v, sem).wait()
        pltpu.sync_copy(rows_v, out_hbm.at[pl.ds(base, b_per_w)])

    raw = gather_pairs(pair_idx, table_pairs)
    odd = (idx & 1)[:, None] == 1
    return jnp.where(odd, raw[:, EMBED_DIM:], raw[:, :EMBED_DIM])


def kernel(user_inputs, table):
    return _embed_lookup(user_inputs, table)


# speedup vs baseline: 1.5780x; 1.5780x over previous
import functools

import jax
import jax.numpy as jnp
from jax import lax
from jax.experimental import pallas as pl
from jax.experimental.pallas import tpu as pltpu
from jax.experimental.pallas import tpu_sc as plsc

NUM_USERS = 1000000
EMBED_DIM = 64
BATCH = 16384
_CHUNK = 64
_L = 16


@jax.jit
def _embed_lookup(user_inputs, table):
    info = plsc.get_sparse_core_info()
    nw = info.num_cores * info.num_subcores
    b_per_w = BATCH // nw
    mesh = plsc.VectorSubcoreMesh(core_axis_name="c", subcore_axis_name="s")

    idx = user_inputs.astype(jnp.int32)

    @functools.partial(
        pl.kernel,
        mesh=mesh,
        out_type=jax.ShapeDtypeStruct((BATCH * EMBED_DIM,), jnp.float32),
        scratch_types=[
            pltpu.VMEM((b_per_w,), jnp.int32),
            pltpu.VMEM((_CHUNK, 8, EMBED_DIM), jnp.float32),
            pltpu.VMEM((b_per_w * EMBED_DIM,), jnp.float32),
            pltpu.SemaphoreType.DMA,
        ],
    )
    def gather_rows(idx_hbm, tbl_hbm, out_hbm, idx_v, blks_v, rows_v, sem):
        wid = lax.axis_index("s") * info.num_cores + lax.axis_index("c")
        base = wid * b_per_w
        pltpu.sync_copy(idx_hbm.at[pl.ds(base, b_per_w)], idx_v)

        for c in range(b_per_w // _CHUNK):
            def fire(grp, carry, c=c):
                v = idx_v[pl.ds(c * _CHUNK + grp * _L, _L)]
                for lane in range(_L):
                    r = v[lane]
                    r8 = pl.multiple_of((r >> 3) << 3, 8)
                    pltpu.async_copy(
                        tbl_hbm.at[pl.ds(r8, 8), :],
                        blks_v.at[grp * _L + lane],
                        sem,
                    )
                return carry

            lax.fori_loop(0, _CHUNK // _L, fire, 0, unroll=False)

            def drain(i, carry):
                pltpu.make_async_copy(
                    tbl_hbm.at[pl.ds(0, 8), :], blks_v.at[i], sem
                ).wait()
                return carry

            lax.fori_loop(0, _CHUNK, drain, 0, unroll=False)

            def select(grp, carry, c=c):
                v = idx_v[pl.ds(c * _CHUNK + grp * _L, _L)]
                for lane in range(_L):
                    g = c * _CHUNK + grp * _L + lane
                    s = v[lane] & 7
                    for q in range(EMBED_DIM // _L):
                        rows_v[pl.ds(g * EMBED_DIM + _L * q, _L)] = blks_v[
                            grp * _L + lane, s, pl.ds(_L * q, _L)
                        ]
                return carry

            lax.fori_loop(0, _CHUNK // _L, select, 0, unroll=False)

        pltpu.sync_copy(
            rows_v, out_hbm.at[pl.ds(base * EMBED_DIM, b_per_w * EMBED_DIM)]
        )

    out_flat = gather_rows(idx, table)
    return out_flat.reshape(BATCH, EMBED_DIM)


def kernel(user_inputs, table):
    return _embed_lookup(user_inputs, table)


# speedup vs baseline: 1.6024x; 1.0155x over previous
import functools

import jax
import jax.numpy as jnp
from jax import lax
from jax.experimental import pallas as pl
from jax.experimental.pallas import tpu as pltpu
from jax.experimental.pallas import tpu_sc as plsc

NUM_USERS = 1000000
EMBED_DIM = 64
BATCH = 16384
_CHUNK = 32
_L = 16


@jax.jit
def _embed_lookup(user_inputs, table):
    info = plsc.get_sparse_core_info()
    nw = info.num_cores * info.num_subcores
    b_per_w = BATCH // nw
    n_chunks = b_per_w // _CHUNK
    mesh = plsc.VectorSubcoreMesh(core_axis_name="c", subcore_axis_name="s")

    idx = user_inputs.astype(jnp.int32)

    @functools.partial(
        pl.kernel,
        mesh=mesh,
        out_type=jax.ShapeDtypeStruct((BATCH * EMBED_DIM,), jnp.float32),
        scratch_types=[
            pltpu.VMEM((b_per_w,), jnp.int32),
            pltpu.VMEM((_CHUNK, 8, EMBED_DIM), jnp.float32),
            pltpu.VMEM((_CHUNK, 8, EMBED_DIM), jnp.float32),
            pltpu.VMEM((b_per_w * EMBED_DIM,), jnp.float32),
            pltpu.SemaphoreType.DMA,
            pltpu.SemaphoreType.DMA,
        ],
    )
    def gather_rows(idx_hbm, tbl_hbm, out_hbm, idx_v, blks_a, blks_b,
                    rows_v, sem_a, sem_b):
        wid = lax.axis_index("s") * info.num_cores + lax.axis_index("c")
        base = wid * b_per_w
        pltpu.sync_copy(idx_hbm.at[pl.ds(base, b_per_w)], idx_v)
        bufs = (blks_a, blks_b)
        sems = (sem_a, sem_b)

        def fire(c, buf, sem):
            def body(grp, carry):
                v = idx_v[pl.ds(c * _CHUNK + grp * _L, _L)]
                for lane in range(_L):
                    r = v[lane]
                    r8 = pl.multiple_of((r >> 3) << 3, 8)
                    pltpu.async_copy(
                        tbl_hbm.at[pl.ds(r8, 8), :],
                        buf.at[grp * _L + lane],
                        sem,
                    )
                return carry

            lax.fori_loop(0, _CHUNK // _L, body, 0, unroll=False)

        def drain_select(c, buf, sem):
            def body(grp, carry):
                for lane in range(_L):
                    pltpu.make_async_copy(
                        tbl_hbm.at[pl.ds(0, 8), :],
                        buf.at[grp * _L + lane],
                        sem,
                    ).wait()
                v = idx_v[pl.ds(c * _CHUNK + grp * _L, _L)]
                for lane in range(_L):
                    g = c * _CHUNK + grp * _L + lane
                    s = v[lane] & 7
                    for q in range(EMBED_DIM // _L):
                        rows_v[pl.ds(g * EMBED_DIM + _L * q, _L)] = buf[
                            grp * _L + lane, s, pl.ds(_L * q, _L)
                        ]
                return carry

            lax.fori_loop(0, _CHUNK // _L, body, 0, unroll=False)

        fire(0, bufs[0], sems[0])
        for c in range(n_chunks):
            if c + 1 < n_chunks:
                fire(c + 1, bufs[(c + 1) % 2], sems[(c + 1) % 2])
            drain_select(c, bufs[c % 2], sems[c % 2])

        pltpu.sync_copy(
            rows_v, out_hbm.at[pl.ds(base * EMBED_DIM, b_per_w * EMBED_DIM)]
        )

    out_flat = gather_rows(idx, table)
    return out_flat.reshape(BATCH, EMBED_DIM)


def kernel(user_inputs, table):
    return _embed_lookup(user_inputs, table)


# speedup vs baseline: 2.2450x; 1.4010x over previous
import functools

import jax
import jax.numpy as jnp
from jax import lax
from jax.experimental import pallas as pl
from jax.experimental.pallas import tpu as pltpu
from jax.experimental.pallas import tpu_sc as plsc

NUM_USERS = 1000000
EMBED_DIM = 64
BATCH = 16384
_CHUNK = 32
_L = 16


@jax.jit
def _embed_lookup(user_inputs, table):
    info = plsc.get_sparse_core_info()
    nw = info.num_cores * info.num_subcores
    b_per_w = BATCH // nw
    n_chunks = b_per_w // _CHUNK
    mesh = plsc.VectorSubcoreMesh(core_axis_name="c", subcore_axis_name="s")

    idx = user_inputs.astype(jnp.int32)
    tbl3 = table.reshape(NUM_USERS // 8, 8, EMBED_DIM)

    @functools.partial(
        pl.kernel,
        mesh=mesh,
        out_type=jax.ShapeDtypeStruct((BATCH * EMBED_DIM,), jnp.float32),
        scratch_types=[
            pltpu.VMEM((b_per_w,), jnp.int32),
            pltpu.VMEM((_CHUNK, 8, EMBED_DIM), jnp.float32),
            pltpu.VMEM((_CHUNK, 8, EMBED_DIM), jnp.float32),
            pltpu.VMEM((b_per_w * EMBED_DIM,), jnp.float32),
            pltpu.SemaphoreType.DMA,
            pltpu.SemaphoreType.DMA,
        ],
    )
    def gather_rows(idx_hbm, tbl_hbm, out_hbm, idx_v, blks_a, blks_b,
                    rows_v, sem_a, sem_b):
        wid = lax.axis_index("s") * info.num_cores + lax.axis_index("c")
        base = wid * b_per_w
        pltpu.sync_copy(idx_hbm.at[pl.ds(base, b_per_w)], idx_v)
        bufs = (blks_a, blks_b)
        sems = (sem_a, sem_b)

        def fire(c, buf, sem):
            def body(grp, carry):
                v = idx_v[pl.ds(c * _CHUNK + grp * _L, _L)]
                for lane in range(_L):
                    j = v[lane] >> 3
                    pltpu.async_copy(
                        tbl_hbm.at[j],
                        buf.at[grp * _L + lane],
                        sem,
                    )
                return carry

            lax.fori_loop(0, _CHUNK // _L, body, 0, unroll=False)

        def drain_select(c, buf, sem):
            def body(grp, carry):
                for lane in range(_L):
                    pltpu.make_async_copy(
                        tbl_hbm.at[0],
                        buf.at[grp * _L + lane],
                        sem,
                    ).wait()
                v = idx_v[pl.ds(c * _CHUNK + grp * _L, _L)]
                for lane in range(_L):
                    g = c * _CHUNK + grp * _L + lane
                    s = v[lane] & 7
                    for q in range(EMBED_DIM // _L):
                        rows_v[pl.ds(g * EMBED_DIM + _L * q, _L)] = buf[
                            grp * _L + lane, s, pl.ds(_L * q, _L)
                        ]
                return carry

            lax.fori_loop(0, _CHUNK // _L, body, 0, unroll=False)

        fire(0, bufs[0], sems[0])
        for c in range(n_chunks):
            if c + 1 < n_chunks:
                fire(c + 1, bufs[(c + 1) % 2], sems[(c + 1) % 2])
            drain_select(c, bufs[c % 2], sems[c % 2])

        pltpu.sync_copy(
            rows_v, out_hbm.at[pl.ds(base * EMBED_DIM, b_per_w * EMBED_DIM)]
        )

    out_flat = gather_rows(idx, tbl3)
    return out_flat.reshape(BATCH, EMBED_DIM)


def kernel(user_inputs, table):
    return _embed_lookup(user_inputs, table)


# speedup vs baseline: 2.2989x; 1.0240x over previous
import functools

import jax
import jax.numpy as jnp
from jax import lax
from jax.experimental import pallas as pl
from jax.experimental.pallas import tpu as pltpu
from jax.experimental.pallas import tpu_sc as plsc

NUM_USERS = 1000000
EMBED_DIM = 64
BATCH = 16384
_CHUNK = 32
_L = 16


@jax.jit
def _embed_lookup(user_inputs, table):
    info = plsc.get_sparse_core_info()
    nw = info.num_cores * info.num_subcores
    b_per_w = BATCH // nw
    n_chunks = b_per_w // _CHUNK
    mesh = plsc.VectorSubcoreMesh(core_axis_name="c", subcore_axis_name="s")

    idx = user_inputs.astype(jnp.int32)
    tbl3 = table.reshape(NUM_USERS // 8, 8, EMBED_DIM)

    @functools.partial(
        pl.kernel,
        mesh=mesh,
        out_type=jax.ShapeDtypeStruct((BATCH // 8, 8, EMBED_DIM), jnp.float32),
        scratch_types=[
            pltpu.VMEM((b_per_w,), jnp.int32),
            pltpu.VMEM((_CHUNK, 8, EMBED_DIM), jnp.float32),
            pltpu.VMEM((_CHUNK, 8, EMBED_DIM), jnp.float32),
            pltpu.VMEM((_CHUNK // 8, 8, EMBED_DIM), jnp.float32),
            pltpu.VMEM((_CHUNK // 8, 8, EMBED_DIM), jnp.float32),
            pltpu.SemaphoreType.DMA,
            pltpu.SemaphoreType.DMA,
            pltpu.SemaphoreType.DMA,
        ],
    )
    def gather_rows(idx_hbm, tbl_hbm, out_hbm, idx_v, blks_a, blks_b,
                    rowsc_a, rowsc_b, sem_a, sem_b, wsem):
        wid = lax.axis_index("s") * info.num_cores + lax.axis_index("c")
        base = wid * b_per_w
        pltpu.sync_copy(idx_hbm.at[pl.ds(base, b_per_w)], idx_v)
        bufs = (blks_a, blks_b)
        sems = (sem_a, sem_b)
        rowsc = (rowsc_a, rowsc_b)

        def fire(c, buf, sem):
            def body(grp, carry):
                v = idx_v[pl.ds(c * _CHUNK + grp * _L, _L)]
                for lane in range(_L):
                    j = v[lane] >> 3
                    pltpu.async_copy(
                        tbl_hbm.at[j],
                        buf.at[grp * _L + lane],
                        sem,
                    )
                return carry

            lax.fori_loop(0, _CHUNK // _L, body, 0, unroll=False)

        def drain_select(c, buf, sem, rc):
            def body(grp, carry):
                for lane in range(_L):
                    pltpu.make_async_copy(
                        tbl_hbm.at[0],
                        buf.at[grp * _L + lane],
                        sem,
                    ).wait()
                v = idx_v[pl.ds(c * _CHUNK + grp * _L, _L)]
                for lane in range(_L):
                    i = grp * _L + lane
                    s = v[lane] & 7
                    for q in range(EMBED_DIM // _L):
                        rc[i >> 3, i & 7, pl.ds(_L * q, _L)] = buf[
                            i, s, pl.ds(_L * q, _L)
                        ]
                return carry

            lax.fori_loop(0, _CHUNK // _L, body, 0, unroll=False)

        fire(0, bufs[0], sems[0])
        for c in range(n_chunks):
            if c + 1 < n_chunks:
                fire(c + 1, bufs[(c + 1) % 2], sems[(c + 1) % 2])
            if c >= 2:
                pltpu.make_async_copy(
                    tbl_hbm.at[pl.ds(0, _CHUNK // 8)], rowsc[c % 2], wsem
                ).wait()
            drain_select(c, bufs[c % 2], sems[c % 2], rowsc[c % 2])
            pltpu.async_copy(
                rowsc[c % 2],
                out_hbm.at[pl.ds(wid * (b_per_w // 8) + c * (_CHUNK // 8),
                                 _CHUNK // 8)],
                wsem,
            )
        for c in (n_chunks - 2, n_chunks - 1):
            pltpu.make_async_copy(
                tbl_hbm.at[pl.ds(0, _CHUNK // 8)], rowsc[c % 2], wsem
            ).wait()

    out3 = gather_rows(idx, tbl3)
    return out3.reshape(BATCH, EMBED_DIM)


def kernel(user_inputs, table):
    return _embed_lookup(user_inputs, table)
